# baseline (speedup 1.0000x reference)
import jax
import jax.numpy as jnp
import numpy as np
from jax.experimental import pallas as pl
from jax.experimental.pallas import tpu as pltpu

WP = 6
WF = 6
ROWB = 128
KWIN = 384


def _edge_att_kernel(lens_ref, nf_ref, w_ref, out_ref, att_ref):
    b = pl.program_id(0)
    r = pl.program_id(1)
    nf = nf_ref[0]
    L = nf.shape[0]
    nt = (((1,), (1,)), ((), ()))

    @pl.when(r == 0)
    def _():
        att_ref[...] = jax.lax.dot_general(
            nf, w_ref[...], nt, preferred_element_type=jnp.float32)

    n = lens_ref[b]
    scale = np.float32(1.0 / np.sqrt(200.0))
    j0 = pl.multiple_of(r * ROWB, ROWB)
    start = pl.multiple_of(
        jnp.minimum(jnp.maximum((r - 1) * ROWB, 0), L - KWIN), ROWB)
    attw = att_ref[pl.ds(start, KWIN), :]
    nfb = nf_ref[0, pl.ds(j0, ROWB), :]
    scores = jax.lax.dot_general(nfb, attw, nt,
                                 preferred_element_type=jnp.float32) * scale
    jj = j0 + jax.lax.broadcasted_iota(jnp.int32, (ROWB, KWIN), 0)
    kk = start + jax.lax.broadcasted_iota(jnp.int32, (ROWB, KWIN), 1)
    mask = (kk >= jj - WP) & (kk <= jj + WF) & (kk < n) & (jj < n)
    e = jnp.where(mask, jnp.exp(scores), jnp.float32(0.0))
    s = jnp.sum(e, axis=1, keepdims=True)
    p = e * jnp.where(s > 0, 1.0 / s, jnp.float32(0.0))
    out_ref[0, :, pl.ds(start, KWIN)] = p
    comp = pl.multiple_of(jnp.where(start == 0, KWIN, 0), L - KWIN)
    out_ref[0, :, pl.ds(comp, L - KWIN)] = jnp.zeros((ROWB, L - KWIN),
                                                     jnp.float32)


def kernel(node_features, node_num_tensor, weight):
    B, L, G = node_features.shape
    lens = node_num_tensor.astype(jnp.int32)
    grid_spec = pltpu.PrefetchScalarGridSpec(
        num_scalar_prefetch=1,
        grid=(B, L // ROWB),
        in_specs=[
            pl.BlockSpec((1, L, G), lambda b, r, lens_ref: (b, 0, 0)),
            pl.BlockSpec((G, G), lambda b, r, lens_ref: (0, 0)),
        ],
        out_specs=pl.BlockSpec((1, ROWB, L), lambda b, r, lens_ref: (b, r, 0)),
        scratch_shapes=[pltpu.VMEM((L, G), jnp.float32)],
    )
    return pl.pallas_call(
        _edge_att_kernel,
        grid_spec=grid_spec,
        out_shape=jax.ShapeDtypeStruct((B, L, L), jnp.float32),
        compiler_params=pltpu.CompilerParams(
            dimension_semantics=("arbitrary", "arbitrary"),
        ),
    )(lens, node_features, weight)


# speedup vs baseline: 1.0108x; 1.0108x over previous
import jax
import jax.numpy as jnp
import numpy as np
from jax.experimental import pallas as pl
from jax.experimental.pallas import tpu as pltpu

WP = 6
WF = 6
ROWB = 128
KWIN = 384


def _edge_att_kernel(lens_ref, nf_ref, w_ref, out_ref, att_ref):
    b = pl.program_id(0)
    r = pl.program_id(1)
    nf = nf_ref[0]
    L = nf.shape[0]
    nt = (((1,), (1,)), ((), ()))

    @pl.when(r == 0)
    def _():
        att_ref[...] = jax.lax.dot_general(
            nf, w_ref[...], nt, preferred_element_type=jnp.float32)

    n = lens_ref[b]
    scale = np.float32(1.0 / np.sqrt(200.0))

    def strip(rr):
        j0 = ROWB * rr
        start = min(max(ROWB * (rr - 1), 0), L - KWIN)
        attw = att_ref[start:start + KWIN, :]
        nfb = nf[j0:j0 + ROWB, :]
        scores = jax.lax.dot_general(
            nfb, attw, nt, preferred_element_type=jnp.float32) * scale
        jj = j0 + jax.lax.broadcasted_iota(jnp.int32, (ROWB, KWIN), 0)
        kk = start + jax.lax.broadcasted_iota(jnp.int32, (ROWB, KWIN), 1)
        mask = (kk >= jj - WP) & (kk <= jj + WF) & (kk < n) & (jj < n)
        e = jnp.where(mask, jnp.exp(scores), jnp.float32(0.0))
        s = jnp.sum(e, axis=1, keepdims=True)
        p = e * jnp.where(s > 0, 1.0 / s, jnp.float32(0.0))
        out_ref[0, :, start:start + KWIN] = p
        comp = KWIN if start == 0 else 0
        out_ref[0, :, comp:comp + (L - KWIN)] = jnp.zeros(
            (ROWB, L - KWIN), jnp.float32)

    for rr in range(L // ROWB):
        pl.when(r == rr)(lambda rr=rr: strip(rr))


def kernel(node_features, node_num_tensor, weight):
    B, L, G = node_features.shape
    lens = node_num_tensor.astype(jnp.int32)
    grid_spec = pltpu.PrefetchScalarGridSpec(
        num_scalar_prefetch=1,
        grid=(B, L // ROWB),
        in_specs=[
            pl.BlockSpec((1, L, G), lambda b, r, lens_ref: (b, 0, 0)),
            pl.BlockSpec((G, G), lambda b, r, lens_ref: (0, 0)),
        ],
        out_specs=pl.BlockSpec((1, ROWB, L), lambda b, r, lens_ref: (b, r, 0)),
        scratch_shapes=[pltpu.VMEM((L, G), jnp.float32)],
    )
    return pl.pallas_call(
        _edge_att_kernel,
        grid_spec=grid_spec,
        out_shape=jax.ShapeDtypeStruct((B, L, L), jnp.float32),
        compiler_params=pltpu.CompilerParams(
            dimension_semantics=("arbitrary", "arbitrary"),
        ),
    )(lens, node_features, weight)


# speedup vs baseline: 3.0684x; 3.0357x over previous
import jax
import jax.numpy as jnp
import numpy as np
from jax.experimental import pallas as pl
from jax.experimental.pallas import tpu as pltpu

WP = 6
WF = 6
ROWB = 128
KWIN = 384
BSTEP = 2


def _edge_att_kernel(lens_ref, nf_ref, w_ref, out_ref):
    bs = pl.program_id(0)
    L = nf_ref.shape[1]
    nt = (((1,), (1,)), ((), ()))
    scale = np.float32(1.0 / np.sqrt(200.0))
    for u in range(BSTEP):
        nf = nf_ref[u]
        att = jax.lax.dot_general(nf, w_ref[...], nt,
                                  preferred_element_type=jnp.float32)
        n = lens_ref[bs * BSTEP + u]
        for r in range(L // ROWB):
            j0 = ROWB * r
            start = min(max(ROWB * (r - 1), 0), L - KWIN)
            scores = jax.lax.dot_general(
                nf[j0:j0 + ROWB], att[start:start + KWIN], nt,
                preferred_element_type=jnp.float32) * scale
            jj = j0 + jax.lax.broadcasted_iota(jnp.int32, (ROWB, KWIN), 0)
            kk = start + jax.lax.broadcasted_iota(jnp.int32, (ROWB, KWIN), 1)
            mask = (kk >= jj - WP) & (kk <= jj + WF) & (kk < n) & (jj < n)
            e = jnp.where(mask, jnp.exp(scores), jnp.float32(0.0))
            s = jnp.sum(e, axis=1, keepdims=True)
            p = e * jnp.where(s > 0, 1.0 / s, jnp.float32(0.0))
            out_ref[u, j0:j0 + ROWB, start:start + KWIN] = p
            comp = KWIN if start == 0 else 0
            out_ref[u, j0:j0 + ROWB, comp:comp + (L - KWIN)] = jnp.zeros(
                (ROWB, L - KWIN), jnp.float32)


def kernel(node_features, node_num_tensor, weight):
    B, L, G = node_features.shape
    lens = node_num_tensor.astype(jnp.int32)
    grid_spec = pltpu.PrefetchScalarGridSpec(
        num_scalar_prefetch=1,
        grid=(B // BSTEP,),
        in_specs=[
            pl.BlockSpec((BSTEP, L, G), lambda b, lens_ref: (b, 0, 0)),
            pl.BlockSpec((G, G), lambda b, lens_ref: (0, 0)),
        ],
        out_specs=pl.BlockSpec((BSTEP, L, L), lambda b, lens_ref: (b, 0, 0)),
    )
    return pl.pallas_call(
        _edge_att_kernel,
        grid_spec=grid_spec,
        out_shape=jax.ShapeDtypeStruct((B, L, L), jnp.float32),
        compiler_params=pltpu.CompilerParams(
            dimension_semantics=("arbitrary",),
        ),
    )(lens, node_features, weight)


# speedup vs baseline: 3.3437x; 1.0897x over previous
import jax
import jax.numpy as jnp
import numpy as np
from jax.experimental import pallas as pl
from jax.experimental.pallas import tpu as pltpu

WP = 6
WF = 6
ROWB = 128
KWIN = 384
BSTEP = 4


def _edge_att_kernel(lens_ref, nf_ref, w_ref, out_ref):
    bs = pl.program_id(0)
    L = nf_ref.shape[1]
    nt = (((1,), (1,)), ((), ()))
    scale = np.float32(1.0 / np.sqrt(200.0))
    for u in range(BSTEP):
        nf = nf_ref[u]
        att = jax.lax.dot_general(nf, w_ref[...], nt,
                                  preferred_element_type=jnp.float32)
        n = lens_ref[bs * BSTEP + u]
        for r in range(L // ROWB):
            j0 = ROWB * r
            start = min(max(ROWB * (r - 1), 0), L - KWIN)
            scores = jax.lax.dot_general(
                nf[j0:j0 + ROWB], att[start:start + KWIN], nt,
                preferred_element_type=jnp.float32) * scale
            jj = j0 + jax.lax.broadcasted_iota(jnp.int32, (ROWB, KWIN), 0)
            kk = start + jax.lax.broadcasted_iota(jnp.int32, (ROWB, KWIN), 1)
            mask = (kk >= jj - WP) & (kk <= jj + WF) & (kk < n) & (jj < n)
            e = jnp.where(mask, jnp.exp(scores), jnp.float32(0.0))
            s = jnp.sum(e, axis=1, keepdims=True)
            p = e * jnp.where(s > 0, 1.0 / s, jnp.float32(0.0))
            out_ref[u, j0:j0 + ROWB, start:start + KWIN] = p
            comp = KWIN if start == 0 else 0
            out_ref[u, j0:j0 + ROWB, comp:comp + (L - KWIN)] = jnp.zeros(
                (ROWB, L - KWIN), jnp.float32)


def kernel(node_features, node_num_tensor, weight):
    B, L, G = node_features.shape
    lens = node_num_tensor.astype(jnp.int32)
    grid_spec = pltpu.PrefetchScalarGridSpec(
        num_scalar_prefetch=1,
        grid=(B // BSTEP,),
        in_specs=[
            pl.BlockSpec((BSTEP, L, G), lambda b, lens_ref: (b, 0, 0)),
            pl.BlockSpec((G, G), lambda b, lens_ref: (0, 0)),
        ],
        out_specs=pl.BlockSpec((BSTEP, L, L), lambda b, lens_ref: (b, 0, 0)),
    )
    return pl.pallas_call(
        _edge_att_kernel,
        grid_spec=grid_spec,
        out_shape=jax.ShapeDtypeStruct((B, L, L), jnp.float32),
        compiler_params=pltpu.CompilerParams(
            dimension_semantics=("arbitrary",),
        ),
    )(lens, node_features, weight)
